# baseline (speedup 1.0000x reference)
import functools
import jax
import jax.numpy as jnp
from jax import lax
from jax.experimental import pallas as pl
from jax.experimental.pallas import tpu as pltpu
from jax.experimental.pallas import tpu_sc as plsc

_B = 4096
_T = 26
_V = 100000
_D = 32
_DF = 128
_H = 512

_NC = 2
_NS = 16
_NW = _NC * _NS


def _sc_gather_body(tab_hbm, idx_hbm, out_hbm, buf_v, idx0_v, idx1_v, out_v,
                    semt, semi0, semi1, semo):
    c = lax.axis_index("c")
    s = lax.axis_index("s")
    w = s * _NC + c

    idx_slot = (idx0_v, idx1_v)
    idx_sem = (semi0, semi1)

    def start_idx(t):
        pltpu.make_async_copy(idx_hbm.at[t], idx_slot[t % 2], idx_sem[t % 2]).start()

    def start_stream(t):
        pltpu.make_async_copy(tab_hbm.at[t, w], buf_v, semt).start()

    start_stream(0)
    start_idx(0)

    for t in range(_T):
        if t + 1 < _T:
            start_idx(t + 1)
        iv = idx_slot[t % 2]
        pltpu.make_async_copy(idx_hbm.at[t], iv, idx_sem[t % 2]).wait()
        if t >= 1:
            pltpu.make_async_copy(out_v, out_hbm.at[(t - 1) * _D + w], semo).wait()
        pltpu.make_async_copy(tab_hbm.at[t, w], buf_v, semt).wait()

        def gather16(k, _, iv=iv):
            for u in range(16):
                sl = pl.ds(k * 256 + u * 16, 16)
                out_v[sl] = plsc.load_gather(buf_v, [iv[sl]])
            return 0

        lax.fori_loop(0, _B // 256, gather16, 0)
        if t + 1 < _T:
            start_stream(t + 1)
        pltpu.make_async_copy(out_v, out_hbm.at[t * _D + w], semo).start()

    pltpu.make_async_copy(out_v, out_hbm.at[(_T - 1) * _D + w], semo).wait()


@jax.jit
def _sc_gather(tab, idx_t):
    mesh = plsc.VectorSubcoreMesh(core_axis_name="c", subcore_axis_name="s")
    return pl.kernel(
        _sc_gather_body,
        out_type=jax.ShapeDtypeStruct((_T * _D, _B), jnp.float32),
        mesh=mesh,
        scratch_types=[
            pltpu.VMEM((_V,), jnp.float32),
            pltpu.VMEM((_B,), jnp.int32),
            pltpu.VMEM((_B,), jnp.int32),
            pltpu.VMEM((_B,), jnp.float32),
            pltpu.SemaphoreType.DMA,
            pltpu.SemaphoreType.DMA,
            pltpu.SemaphoreType.DMA,
            pltpu.SemaphoreType.DMA,
        ],
        compiler_params=pltpu.CompilerParams(
            use_tc_tiling_on_sc=True, needs_layout_passes=False),
    )(tab, idx_t)


def _dense_body(df_ref, w1_ref, b1_ref, w2_ref, b2_ref, dx_ref):
    f32 = jnp.float32
    h = jnp.maximum(
        jnp.dot(df_ref[...], w1_ref[...], preferred_element_type=f32) + b1_ref[...], 0.0)
    dx_ref[...] = jnp.maximum(
        jnp.dot(h, w2_ref[...], preferred_element_type=f32) + b2_ref[...], 0.0)


@jax.jit
def _tc_dense(df, w1, b1, w2, b2):
    full = lambda shape: pl.BlockSpec(shape, lambda i: (0, 0))
    return pl.pallas_call(
        _dense_body,
        grid=(4,),
        in_specs=[
            pl.BlockSpec((_B // 4, _DF), lambda i: (i, 0)),
            full((_DF, _DF)),
            full((1, _DF)),
            full((_DF, _D)),
            full((1, _D)),
        ],
        out_specs=pl.BlockSpec((_B // 4, _D), lambda i: (i, 0)),
        out_shape=jax.ShapeDtypeStruct((_B, _D), jnp.float32),
        compiler_params=pltpu.CompilerParams(
            dimension_semantics=("arbitrary",),
        ),
    )(df, w1, b1, w2, b2)


def _mlp_body(dx_ref, embt_ref, w3_ref, b3_ref, w4_ref, b4_ref, out_ref):
    f32 = jnp.float32
    dx = dx_ref[...]
    emb_w3 = lax.dot_general(
        embt_ref[...], w3_ref[_D:, :],
        dimension_numbers=(((0,), (0,)), ((), ())),
        preferred_element_type=f32)
    g = (jnp.dot(dx, w3_ref[0:_D, :], preferred_element_type=f32)
         + emb_w3 + b3_ref[...])
    g = jnp.maximum(g, 0.0)
    out_ref[...] = jnp.maximum(
        jnp.dot(g, w4_ref[...], preferred_element_type=f32) + b4_ref[...], 0.0)


@functools.partial(jax.jit, static_argnames=("bb",))
def _tc_mlp(dx, embt, w3, b3, w4, b4, bb=1024):
    grid = (_B // bb,)
    full = lambda shape: pl.BlockSpec(shape, lambda i: (0, 0))
    return pl.pallas_call(
        _mlp_body,
        grid=grid,
        in_specs=[
            pl.BlockSpec((bb, _D), lambda i: (i, 0)),
            pl.BlockSpec((_T * _D, bb), lambda i: (0, i)),
            full((_D + _T * _D, _H)),
            full((1, _H)),
            full((_H, 1)),
            full((1, 1)),
        ],
        out_specs=pl.BlockSpec((bb, 1), lambda i: (i, 0)),
        out_shape=jax.ShapeDtypeStruct((_B, 1), jnp.float32),
        compiler_params=pltpu.CompilerParams(
            dimension_semantics=("arbitrary",),
        ),
    )(dx, embt, w3, b3, w4, b4)


def kernel(dense_features, sharded_sparse_features, tables, w1, b1, w2, b2, w3, b3, w4, b4):
    tt = jnp.transpose(tables, (0, 2, 1))
    idx_t = sharded_sparse_features.astype(jnp.int32).T
    embt = _sc_gather(tt, idx_t)
    dx = _tc_dense(dense_features, w1, b1.reshape(1, _DF), w2, b2.reshape(1, _D))
    return _tc_mlp(dx, embt, w3, b3.reshape(1, _H), w4, b4.reshape(1, 1))


# speedup vs baseline: 1.0093x; 1.0093x over previous
import functools
import jax
import jax.numpy as jnp
from jax import lax
from jax.experimental import pallas as pl
from jax.experimental.pallas import tpu as pltpu
from jax.experimental.pallas import tpu_sc as plsc

_B = 4096
_T = 26
_V = 100000
_D = 32
_DF = 128
_H = 512

_NC = 2
_NS = 16
_NW = _NC * _NS


def _sc_gather_body(tab_hbm, idx_hbm, out_hbm, buf_v, idx0_v, idx1_v, out_v,
                    semt, semi0, semi1, semo):
    c = lax.axis_index("c")
    s = lax.axis_index("s")
    w = s * _NC + c

    idx_slot = (idx0_v, idx1_v)
    idx_sem = (semi0, semi1)

    def start_idx(t):
        pltpu.make_async_copy(idx_hbm.at[t], idx_slot[t % 2], idx_sem[t % 2]).start()

    def start_stream(t):
        pltpu.make_async_copy(tab_hbm.at[t, w], buf_v, semt).start()

    start_stream(0)
    start_idx(0)

    for t in range(_T):
        if t + 1 < _T:
            start_idx(t + 1)
        iv = idx_slot[t % 2]
        pltpu.make_async_copy(idx_hbm.at[t], iv, idx_sem[t % 2]).wait()
        if t >= 1:
            pltpu.make_async_copy(out_v, out_hbm.at[(t - 1) * _D + w], semo).wait()
        pltpu.make_async_copy(tab_hbm.at[t, w], buf_v, semt).wait()

        def gather8(k, _, iv=iv):
            for u in range(8):
                sl = pl.ds(k * 128 + u * 16, 16)
                out_v[sl] = plsc.load_gather(buf_v, [iv[sl]])
            return 0

        lax.fori_loop(0, _B // 128, gather8, 0)
        if t + 1 < _T:
            start_stream(t + 1)
        pltpu.make_async_copy(out_v, out_hbm.at[t * _D + w], semo).start()

    pltpu.make_async_copy(out_v, out_hbm.at[(_T - 1) * _D + w], semo).wait()


@jax.jit
def _sc_gather(tab, idx_t):
    mesh = plsc.VectorSubcoreMesh(core_axis_name="c", subcore_axis_name="s")
    return pl.kernel(
        _sc_gather_body,
        out_type=jax.ShapeDtypeStruct((_T * _D, _B), jnp.float32),
        mesh=mesh,
        scratch_types=[
            pltpu.VMEM((_V,), jnp.float32),
            pltpu.VMEM((_B,), jnp.int32),
            pltpu.VMEM((_B,), jnp.int32),
            pltpu.VMEM((_B,), jnp.float32),
            pltpu.SemaphoreType.DMA,
            pltpu.SemaphoreType.DMA,
            pltpu.SemaphoreType.DMA,
            pltpu.SemaphoreType.DMA,
        ],
        compiler_params=pltpu.CompilerParams(
            use_tc_tiling_on_sc=True, needs_layout_passes=False),
    )(tab, idx_t)


def _dense_body(df_ref, w1_ref, b1_ref, w2_ref, b2_ref, dx_ref):
    f32 = jnp.float32
    h = jnp.maximum(
        jnp.dot(df_ref[...], w1_ref[...], preferred_element_type=f32) + b1_ref[...], 0.0)
    dx_ref[...] = jnp.maximum(
        jnp.dot(h, w2_ref[...], preferred_element_type=f32) + b2_ref[...], 0.0)


@jax.jit
def _tc_dense(df, w1, b1, w2, b2):
    full = lambda shape: pl.BlockSpec(shape, lambda i: (0, 0))
    return pl.pallas_call(
        _dense_body,
        grid=(4,),
        in_specs=[
            pl.BlockSpec((_B // 4, _DF), lambda i: (i, 0)),
            full((_DF, _DF)),
            full((1, _DF)),
            full((_DF, _D)),
            full((1, _D)),
        ],
        out_specs=pl.BlockSpec((_B // 4, _D), lambda i: (i, 0)),
        out_shape=jax.ShapeDtypeStruct((_B, _D), jnp.float32),
        compiler_params=pltpu.CompilerParams(
            dimension_semantics=("arbitrary",),
        ),
    )(df, w1, b1, w2, b2)


def _mlp_body(dx_ref, embt_ref, w3_ref, b3_ref, w4_ref, b4_ref, out_ref):
    f32 = jnp.float32
    dx = dx_ref[...]
    emb_w3 = lax.dot_general(
        embt_ref[...], w3_ref[_D:, :],
        dimension_numbers=(((0,), (0,)), ((), ())),
        preferred_element_type=f32)
    g = (jnp.dot(dx, w3_ref[0:_D, :], preferred_element_type=f32)
         + emb_w3 + b3_ref[...])
    g = jnp.maximum(g, 0.0)
    out_ref[...] = jnp.maximum(
        jnp.dot(g, w4_ref[...], preferred_element_type=f32) + b4_ref[...], 0.0)


@functools.partial(jax.jit, static_argnames=("bb",))
def _tc_mlp(dx, embt, w3, b3, w4, b4, bb=1024):
    grid = (_B // bb,)
    full = lambda shape: pl.BlockSpec(shape, lambda i: (0, 0))
    return pl.pallas_call(
        _mlp_body,
        grid=grid,
        in_specs=[
            pl.BlockSpec((bb, _D), lambda i: (i, 0)),
            pl.BlockSpec((_T * _D, bb), lambda i: (0, i)),
            full((_D + _T * _D, _H)),
            full((1, _H)),
            full((_H, 1)),
            full((1, 1)),
        ],
        out_specs=pl.BlockSpec((bb, 1), lambda i: (i, 0)),
        out_shape=jax.ShapeDtypeStruct((_B, 1), jnp.float32),
        compiler_params=pltpu.CompilerParams(
            dimension_semantics=("arbitrary",),
        ),
    )(dx, embt, w3, b3, w4, b4)


def kernel(dense_features, sharded_sparse_features, tables, w1, b1, w2, b2, w3, b3, w4, b4):
    tt = jnp.transpose(tables, (0, 2, 1))
    idx_t = sharded_sparse_features.astype(jnp.int32).T
    embt = _sc_gather(tt, idx_t)
    dx = _tc_dense(dense_features, w1, b1.reshape(1, _DF), w2, b2.reshape(1, _D))
    return _tc_mlp(dx, embt, w3, b3.reshape(1, _H), w4, b4.reshape(1, 1))


# speedup vs baseline: 1.0130x; 1.0037x over previous
import functools
import jax
import jax.numpy as jnp
from jax import lax
from jax.experimental import pallas as pl
from jax.experimental.pallas import tpu as pltpu
from jax.experimental.pallas import tpu_sc as plsc

_B = 4096
_T = 26
_V = 100000
_D = 32
_DF = 128
_H = 512

_NC = 2
_NS = 16
_NW = _NC * _NS


def _sc_gather_body(tab_hbm, idx_hbm, out_hbm, buf_v, idx0_v, idx1_v, out_v,
                    semt, semi0, semi1, semo):
    c = lax.axis_index("c")
    s = lax.axis_index("s")
    w = s * _NC + c

    idx_slot = (idx0_v, idx1_v)
    idx_sem = (semi0, semi1)

    def start_idx(t):
        pltpu.make_async_copy(idx_hbm.at[t], idx_slot[t % 2], idx_sem[t % 2]).start()

    def start_stream(t):
        pltpu.make_async_copy(tab_hbm.at[t, w], buf_v, semt).start()

    start_stream(0)
    start_idx(0)

    for t in range(_T):
        if t + 1 < _T:
            start_idx(t + 1)
        iv = idx_slot[t % 2]
        pltpu.make_async_copy(idx_hbm.at[t], iv, idx_sem[t % 2]).wait()
        if t >= 1:
            pltpu.make_async_copy(out_v, out_hbm.at[(t - 1) * _D + w], semo).wait()
        pltpu.make_async_copy(tab_hbm.at[t, w], buf_v, semt).wait()

        def gather8(k, _, iv=iv):
            for u in range(8):
                sl = pl.ds(k * 128 + u * 16, 16)
                out_v[sl] = plsc.load_gather(buf_v, [iv[sl]])
            return 0

        lax.fori_loop(0, _B // 128, gather8, 0)
        if t + 1 < _T:
            start_stream(t + 1)
        pltpu.make_async_copy(out_v, out_hbm.at[t * _D + w], semo).start()

    pltpu.make_async_copy(out_v, out_hbm.at[(_T - 1) * _D + w], semo).wait()


@jax.jit
def _sc_gather(tab, idx_t):
    mesh = plsc.VectorSubcoreMesh(core_axis_name="c", subcore_axis_name="s")
    return pl.kernel(
        _sc_gather_body,
        out_type=jax.ShapeDtypeStruct((_T * _D, _B), jnp.float32),
        mesh=mesh,
        scratch_types=[
            pltpu.VMEM((_V,), jnp.float32),
            pltpu.VMEM((_B,), jnp.int32),
            pltpu.VMEM((_B,), jnp.int32),
            pltpu.VMEM((_B,), jnp.float32),
            pltpu.SemaphoreType.DMA,
            pltpu.SemaphoreType.DMA,
            pltpu.SemaphoreType.DMA,
            pltpu.SemaphoreType.DMA,
        ],
        compiler_params=pltpu.CompilerParams(
            use_tc_tiling_on_sc=True, needs_layout_passes=False),
    )(tab, idx_t)


def _dense_body(df_ref, w1_ref, b1_ref, w2_ref, b2_ref, dx_ref):
    f32 = jnp.float32
    h = jnp.maximum(
        jnp.dot(df_ref[...], w1_ref[...], preferred_element_type=f32) + b1_ref[...], 0.0)
    dx_ref[...] = jnp.maximum(
        jnp.dot(h, w2_ref[...], preferred_element_type=f32) + b2_ref[...], 0.0)


@jax.jit
def _tc_dense(df, w1, b1, w2, b2):
    full = lambda shape: pl.BlockSpec(shape, lambda i: (0, 0))
    return pl.pallas_call(
        _dense_body,
        grid=(4,),
        in_specs=[
            pl.BlockSpec((_B // 4, _DF), lambda i: (i, 0)),
            full((_DF, _DF)),
            full((1, _DF)),
            full((_DF, _D)),
            full((1, _D)),
        ],
        out_specs=pl.BlockSpec((_B // 4, _D), lambda i: (i, 0)),
        out_shape=jax.ShapeDtypeStruct((_B, _D), jnp.float32),
        compiler_params=pltpu.CompilerParams(
            dimension_semantics=("arbitrary",),
        ),
    )(df, w1, b1, w2, b2)


def _mlp_body(dx_ref, embt_ref, w3_ref, b3_ref, w4_ref, b4_ref, out_ref):
    f32 = jnp.float32
    dx = dx_ref[...]
    emb_w3 = lax.dot_general(
        embt_ref[...].astype(jnp.bfloat16), w3_ref[_D:, :].astype(jnp.bfloat16),
        dimension_numbers=(((0,), (0,)), ((), ())),
        preferred_element_type=f32)
    g = (jnp.dot(dx, w3_ref[0:_D, :], preferred_element_type=f32)
         + emb_w3 + b3_ref[...])
    g = jnp.maximum(g, 0.0)
    out_ref[...] = jnp.maximum(
        jnp.dot(g, w4_ref[...], preferred_element_type=f32) + b4_ref[...], 0.0)


@functools.partial(jax.jit, static_argnames=("bb",))
def _tc_mlp(dx, embt, w3, b3, w4, b4, bb=1024):
    grid = (_B // bb,)
    full = lambda shape: pl.BlockSpec(shape, lambda i: (0, 0))
    return pl.pallas_call(
        _mlp_body,
        grid=grid,
        in_specs=[
            pl.BlockSpec((bb, _D), lambda i: (i, 0)),
            pl.BlockSpec((_T * _D, bb), lambda i: (0, i)),
            full((_D + _T * _D, _H)),
            full((1, _H)),
            full((_H, 1)),
            full((1, 1)),
        ],
        out_specs=pl.BlockSpec((bb, 1), lambda i: (i, 0)),
        out_shape=jax.ShapeDtypeStruct((_B, 1), jnp.float32),
        compiler_params=pltpu.CompilerParams(
            dimension_semantics=("arbitrary",),
        ),
    )(dx, embt, w3, b3, w4, b4)


def kernel(dense_features, sharded_sparse_features, tables, w1, b1, w2, b2, w3, b3, w4, b4):
    tt = jnp.transpose(tables, (0, 2, 1))
    idx_t = sharded_sparse_features.astype(jnp.int32).T
    embt = _sc_gather(tt, idx_t)
    dx = _tc_dense(dense_features, w1, b1.reshape(1, _DF), w2, b2.reshape(1, _D))
    return _tc_mlp(dx, embt, w3, b3.reshape(1, _H), w4, b4.reshape(1, 1))


# speedup vs baseline: 1.0161x; 1.0031x over previous
import functools
import jax
import jax.numpy as jnp
from jax import lax
from jax.experimental import pallas as pl
from jax.experimental.pallas import tpu as pltpu
from jax.experimental.pallas import tpu_sc as plsc

_B = 4096
_T = 26
_V = 100000
_D = 32
_DF = 128
_H = 512

_NC = 2
_NS = 16
_NW = _NC * _NS


def _sc_gather_body(tab_hbm, idx_hbm, out_hbm, buf_v, idx0_v, idx1_v, out_v,
                    semt, semi0, semi1, semo):
    c = lax.axis_index("c")
    s = lax.axis_index("s")
    w = s * _NC + c

    idx_slot = (idx0_v, idx1_v)
    idx_sem = (semi0, semi1)

    def start_idx(t):
        pltpu.make_async_copy(idx_hbm.at[t], idx_slot[t % 2], idx_sem[t % 2]).start()

    def start_stream(t):
        pltpu.make_async_copy(tab_hbm.at[t, w], buf_v, semt).start()

    start_stream(0)
    start_idx(0)

    for t in range(_T):
        if t + 1 < _T:
            start_idx(t + 1)
        iv = idx_slot[t % 2]
        pltpu.make_async_copy(idx_hbm.at[t], iv, idx_sem[t % 2]).wait()
        if t >= 1:
            pltpu.make_async_copy(out_v, out_hbm.at[(t - 1) * _D + w], semo).wait()
        pltpu.make_async_copy(tab_hbm.at[t, w], buf_v, semt).wait()

        def gather8(k, _, iv=iv):
            for u in range(8):
                sl = pl.ds(k * 128 + u * 16, 16)
                out_v[sl] = plsc.load_gather(buf_v, [iv[sl]])
            return 0

        lax.fori_loop(0, _B // 128, gather8, 0)
        if t + 1 < _T:
            start_stream(t + 1)
        pltpu.make_async_copy(out_v, out_hbm.at[t * _D + w], semo).start()

    pltpu.make_async_copy(out_v, out_hbm.at[(_T - 1) * _D + w], semo).wait()


@jax.jit
def _sc_gather(tab, idx_t):
    mesh = plsc.VectorSubcoreMesh(core_axis_name="c", subcore_axis_name="s")
    return pl.kernel(
        _sc_gather_body,
        out_type=jax.ShapeDtypeStruct((_T * _D, _B), jnp.float32),
        mesh=mesh,
        scratch_types=[
            pltpu.VMEM((_V,), jnp.float32),
            pltpu.VMEM((_B,), jnp.int32),
            pltpu.VMEM((_B,), jnp.int32),
            pltpu.VMEM((_B,), jnp.float32),
            pltpu.SemaphoreType.DMA,
            pltpu.SemaphoreType.DMA,
            pltpu.SemaphoreType.DMA,
            pltpu.SemaphoreType.DMA,
        ],
        compiler_params=pltpu.CompilerParams(
            use_tc_tiling_on_sc=True, needs_layout_passes=False),
    )(tab, idx_t)


def _dense_body(df_ref, w1_ref, b1_ref, w2_ref, b2_ref, dx_ref):
    f32 = jnp.float32
    h = jnp.maximum(
        jnp.dot(df_ref[...], w1_ref[...], preferred_element_type=f32) + b1_ref[...], 0.0)
    dx_ref[...] = jnp.maximum(
        jnp.dot(h, w2_ref[...], preferred_element_type=f32) + b2_ref[...], 0.0)


@jax.jit
def _tc_dense(df, w1, b1, w2, b2):
    full = lambda shape: pl.BlockSpec(shape, lambda i: (0, 0))
    return pl.pallas_call(
        _dense_body,
        grid=(4,),
        in_specs=[
            pl.BlockSpec((_B // 4, _DF), lambda i: (i, 0)),
            full((_DF, _DF)),
            full((1, _DF)),
            full((_DF, _D)),
            full((1, _D)),
        ],
        out_specs=pl.BlockSpec((_B // 4, _D), lambda i: (i, 0)),
        out_shape=jax.ShapeDtypeStruct((_B, _D), jnp.float32),
        compiler_params=pltpu.CompilerParams(
            dimension_semantics=("arbitrary",),
        ),
    )(df, w1, b1, w2, b2)


def _mlp_body(dx_ref, embt_ref, w3_ref, b3_ref, w4_ref, b4_ref, out_ref):
    f32 = jnp.float32
    dx = dx_ref[...]
    emb_w3 = lax.dot_general(
        embt_ref[...], w3_ref[_D:, :],
        dimension_numbers=(((0,), (0,)), ((), ())),
        preferred_element_type=f32)
    g = (jnp.dot(dx, w3_ref[0:_D, :], preferred_element_type=f32)
         + emb_w3 + b3_ref[...])
    g = jnp.maximum(g, 0.0)
    out_ref[...] = jnp.maximum(
        jnp.dot(g, w4_ref[...], preferred_element_type=f32) + b4_ref[...], 0.0)


@functools.partial(jax.jit, static_argnames=("bb",))
def _tc_mlp(dx, embt, w3, b3, w4, b4, bb=1024):
    grid = (_B // bb,)
    full = lambda shape: pl.BlockSpec(shape, lambda i: (0, 0))
    return pl.pallas_call(
        _mlp_body,
        grid=grid,
        in_specs=[
            pl.BlockSpec((bb, _D), lambda i: (i, 0)),
            pl.BlockSpec((_T * _D, bb), lambda i: (0, i)),
            full((_D + _T * _D, _H)),
            full((1, _H)),
            full((_H, 1)),
            full((1, 1)),
        ],
        out_specs=pl.BlockSpec((bb, 1), lambda i: (i, 0)),
        out_shape=jax.ShapeDtypeStruct((_B, 1), jnp.float32),
        compiler_params=pltpu.CompilerParams(
            dimension_semantics=("arbitrary",),
        ),
    )(dx, embt, w3, b3, w4, b4)


def kernel(dense_features, sharded_sparse_features, tables, w1, b1, w2, b2, w3, b3, w4, b4):
    tt = jnp.transpose(tables, (0, 2, 1))
    idx_t = sharded_sparse_features.astype(jnp.int32).T
    embt = _sc_gather(tt, idx_t)
    dx = _tc_dense(dense_features, w1, b1.reshape(1, _DF), w2, b2.reshape(1, _D))
    return _tc_mlp(dx, embt, w3, b3.reshape(1, _H), w4, b4.reshape(1, 1))
